# baseline (speedup 1.0000x reference)
import functools

import jax
import jax.numpy as jnp
from jax import lax
from jax.experimental import pallas as pl
from jax.experimental.pallas import tpu as pltpu
from jax.experimental.pallas import tpu_sc as plsc

BATCH = 4096
SEQ = 200
HIDDEN = 64
NUM_WORKERS = 32
IDX_W = 100
CHUNK_IDX_ROWS = 8
CHUNK = IDX_W * CHUNK_IDX_ROWS
TOTAL_ROWS = BATCH * SEQ
ROWS_PER_WORKER = TOTAL_ROWS // NUM_WORKERS
CHUNKS_PER_WORKER = ROWS_PER_WORKER // CHUNK
LANES = 16
H_GROUPS = HIDDEN // LANES


def _sc_body(idx_ref, table_ref, pos_ref, out_ref, idx_v, pos_v, rows_v, sem):
    nc = 2
    wid = lax.axis_index("s") * nc + lax.axis_index("c")
    row_base = wid * ROWS_PER_WORKER
    idx_row_base = row_base // IDX_W

    pltpu.sync_copy(pos_ref, pos_v)

    @pl.loop(0, CHUNKS_PER_WORKER)
    def _chunk(g):
        r0 = pl.multiple_of(row_base + g * CHUNK, CHUNK)
        ir0 = pl.multiple_of(idx_row_base + g * CHUNK_IDX_ROWS, CHUNK_IDX_ROWS)
        pltpu.sync_copy(idx_ref.at[pl.ds(ir0, CHUNK_IDX_ROWS)], idx_v)
        copies = []
        for j in range(CHUNK_IDX_ROWS):
            copies.append(pltpu.async_copy(
                table_ref.at[idx_v.at[j]],
                rows_v.at[pl.ds(j * IDX_W, IDX_W)], sem))
        for c in copies:
            c.wait()

        @pl.loop(0, SEQ)
        def _add(r):
            for half in range(CHUNK // SEQ):
                rr = r + half * SEQ
                for cgrp in range(H_GROUPS):
                    sl = pl.ds(cgrp * LANES, LANES)
                    rows_v[rr, sl] = rows_v[rr, sl] + pos_v[r, sl]

        pltpu.sync_copy(rows_v, out_ref.at[pl.ds(r0, CHUNK)])


@functools.partial(jax.jit, static_argnames=())
def _embed(idx2d, table, pos):
    mesh = plsc.VectorSubcoreMesh(core_axis_name="c", subcore_axis_name="s")
    f = pl.kernel(
        _sc_body,
        out_type=jax.ShapeDtypeStruct((TOTAL_ROWS, HIDDEN), jnp.float32),
        mesh=mesh,
        scratch_types=[
            pltpu.VMEM((CHUNK_IDX_ROWS, IDX_W), jnp.int32),
            pltpu.VMEM((SEQ, HIDDEN), jnp.float32),
            pltpu.VMEM((CHUNK, HIDDEN), jnp.float32),
            pltpu.SemaphoreType.DMA,
        ],
        compiler_params=pltpu.CompilerParams(use_tc_tiling_on_sc=False),
    )
    return f(idx2d, table, pos)


def kernel(input_ids, word_embeddings, position_embeddings):
    idx2d = input_ids.reshape(TOTAL_ROWS // IDX_W, IDX_W).astype(jnp.int32)
    pos = position_embeddings[:SEQ]
    out = _embed(idx2d, word_embeddings, pos)
    return out.reshape(BATCH, SEQ, HIDDEN)
